# baseline (speedup 1.0000x reference)
import functools

import jax
import jax.numpy as jnp
from jax.experimental import pallas as pl
from jax.experimental.pallas import tpu as pltpu

N = 100000
E = 1600000
D = 128
T = 12
B = 2000
H = 128

K = 256
NB = (N + K - 1) // K
L = 2560
TP = 16
SL = 16
SLP = 24


def _body(off_ref, dest_ref, fl_ref, off8_ref,
          x_ref, r_ref, a_ref, ab_ref,
          fc1w_ref, fc1b_ref, fc2w_ref, fc2b_ref, fc3w_ref, fc3b_ref,
          outw_ref, outb_ref,
          out_ref, mol_scr, c_scr):
    i = pl.program_id(0)

    @pl.when(i == 0)
    def _init():
        mol_scr[...] = jnp.zeros_like(mol_scr)

    row = i * K + jax.lax.broadcasted_iota(jnp.int32, (K, 1), 0)
    x = jnp.where(row < N, x_ref[...], 0.0)
    s = jax.nn.sigmoid(x @ a_ref[...] + ab_ref[...])
    s = jnp.where(row < N, s, 0.0)

    y = jnp.concatenate([s[:, t:t + 1] * x for t in range(T)], axis=1)
    y = y.astype(jnp.bfloat16)

    r_row = r_ref[0]
    jcol = jax.lax.broadcasted_iota(jnp.int32, (K, 1), 0)
    m_t = (jcol == r_row).astype(jnp.bfloat16)
    c_scr[...] = jnp.dot(m_t, y, preferred_element_type=jnp.float32)

    base = off_ref[i]
    nd = off_ref[i + 1] - base

    @pl.when(fl_ref[i] == 1)
    def _fast():
        b8 = pl.multiple_of(off8_ref[i] * 8, 8)
        mol_scr[pl.ds(b8, SLP), :] += c_scr[:SLP, :]

    @pl.when(fl_ref[i] == 0)
    def _slow():
        def sbody(j, _):
            d = dest_ref[base + j]
            mol_scr[pl.ds(d, 1), :] += c_scr[pl.ds(j, 1), :]
            return 0

        jax.lax.fori_loop(0, nd, sbody, 0)

    @pl.when(i == NB - 1)
    def _heads():
        for t in range(T):
            mt = mol_scr[:B, t * D:(t + 1) * D]
            h = jnp.maximum(mt @ fc1w_ref[t] + fc1b_ref[t:t + 1, :], 0.0)
            h = jnp.maximum(h @ fc2w_ref[t] + fc2b_ref[t:t + 1, :], 0.0)
            h = jnp.maximum(h @ fc3w_ref[t] + fc3b_ref[t:t + 1, :], 0.0)
            p = h @ outw_ref[..., t:t + 1] + outb_ref[0, t]
            out_ref[:, t:t + 1] = p


@functools.partial(jax.jit, static_argnames=())
def kernel(node_feats, etype, graph_ids, atom_W, atom_b, fc1_W, fc1_b,
           fc2_W, fc2_b, fc3_W, fc3_b, out_W, out_b):
    del etype
    g = graph_ids.astype(jnp.int32)

    idx = jnp.arange(N, dtype=jnp.int32)
    first = jnp.concatenate(
        [jnp.ones((1,), jnp.bool_), g[1:] != g[:-1]]) | ((idx % K) == 0)
    slot = jnp.cumsum(first.astype(jnp.int32)) - 1
    off_blocks = slot[0::K]
    total = slot[-1] + 1
    off = jnp.concatenate([off_blocks, total[None]]).astype(jnp.int32)
    r = slot - jnp.repeat(off_blocks, K, total_repeat_length=NB * K)[:N]
    dest = g[jnp.clip(jnp.searchsorted(slot, jnp.arange(L, dtype=jnp.int32),
                                       side="left"), 0, N - 1)]
    nd_b = off[1:] - off[:-1]
    bad = (dest[1:] - dest[:-1]) != 1
    cb = jnp.concatenate([jnp.zeros((1,), jnp.int32),
                          jnp.cumsum(bad.astype(jnp.int32))])
    anybad = cb[jnp.maximum(off[1:] - 1, off[:-1])] - cb[off[:-1]]
    fl = ((anybad == 0) & (nd_b <= SL)).astype(jnp.int32)
    d0_b = dest[off[:-1]]
    d0a_b = (d0_b // 8) * 8
    q_b = jnp.where(fl == 1, d0_b - d0a_b, 0)
    off8 = (d0a_b // 8).astype(jnp.int32)
    r = r + jnp.repeat(q_b, K, total_repeat_length=NB * K)[:N]
    r3 = jnp.full((NB * K,), -1, jnp.int32).at[:N].set(r).reshape(NB, 1, K)

    a_pad = jnp.zeros((D, TP), jnp.float32).at[:, :T].set(atom_W[:, :, 0].T)
    ab_pad = jnp.zeros((1, TP), jnp.float32).at[0, :T].set(atom_b[:, 0])
    outw = out_W[:, :, 0].T
    outb = out_b[:, 0][None, :]

    grid_spec = pltpu.PrefetchScalarGridSpec(
        num_scalar_prefetch=4,
        grid=(NB,),
        in_specs=[
            pl.BlockSpec((K, D), lambda i, *_: (i, 0)),
            pl.BlockSpec((1, 1, K), lambda i, *_: (i, 0, 0)),
            pl.BlockSpec((D, TP), lambda i, *_: (0, 0)),
            pl.BlockSpec((1, TP), lambda i, *_: (0, 0)),
            pl.BlockSpec((T, D, H), lambda i, *_: (0, 0, 0)),
            pl.BlockSpec((T, H), lambda i, *_: (0, 0)),
            pl.BlockSpec((T, H, H), lambda i, *_: (0, 0, 0)),
            pl.BlockSpec((T, H), lambda i, *_: (0, 0)),
            pl.BlockSpec((T, H, H), lambda i, *_: (0, 0, 0)),
            pl.BlockSpec((T, H), lambda i, *_: (0, 0)),
            pl.BlockSpec((H, T), lambda i, *_: (0, 0)),
            pl.BlockSpec((1, T), lambda i, *_: (0, 0)),
        ],
        out_specs=pl.BlockSpec((B, T), lambda i, *_: (0, 0)),
        scratch_shapes=[
            pltpu.VMEM((B + SLP, T * D), jnp.float32),
            pltpu.VMEM((K, T * D), jnp.float32),
        ],
    )
    return pl.pallas_call(
        _body,
        grid_spec=grid_spec,
        out_shape=jax.ShapeDtypeStruct((B, T), jnp.float32),
        compiler_params=pltpu.CompilerParams(
            dimension_semantics=("arbitrary",)),
    )(off, dest, fl, off8, node_feats, r3, a_pad, ab_pad,
      fc1_W, fc1_b, fc2_W, fc2_b, fc3_W, fc3_b, outw, outb)


# speedup vs baseline: 2.9434x; 2.9434x over previous
import functools

import jax
import jax.numpy as jnp
from jax.experimental import pallas as pl
from jax.experimental.pallas import tpu as pltpu

N = 100000
E = 1600000
D = 128
T = 12
B = 2000
H = 128

K = 256
NB = (N + K - 1) // K
L = 2560
TP = 16
SL = 16
SLP = 24


def _body(off_ref, dest_ref, fl_ref, off8_ref,
          x_ref, r_ref, a_ref, ab_ref,
          fc1w_ref, fc1b_ref, fc2w_ref, fc2b_ref, fc3w_ref, fc3b_ref,
          outw_ref, outb_ref,
          out_ref, mol_scr, c_scr):
    i = pl.program_id(0)

    @pl.when(i == 0)
    def _init():
        mol_scr[...] = jnp.zeros_like(mol_scr)

    row = i * K + jax.lax.broadcasted_iota(jnp.int32, (K, 1), 0)
    x = jnp.where(row < N, x_ref[...], 0.0)
    s = jax.nn.sigmoid(x @ a_ref[...] + ab_ref[...])
    s = jnp.where(row < N, s, 0.0)

    y = jnp.concatenate([s[:, t:t + 1] * x for t in range(T)], axis=1)
    y = y.astype(jnp.bfloat16)

    r_row = r_ref[0]
    jcol = jax.lax.broadcasted_iota(jnp.int32, (K, 1), 0)
    m_t = (jcol == r_row).astype(jnp.bfloat16)
    c_scr[...] = jnp.dot(m_t, y, preferred_element_type=jnp.float32)

    base = off_ref[i]
    nd = off_ref[i + 1] - base

    @pl.when(fl_ref[i] == 1)
    def _fast():
        b8 = pl.multiple_of(off8_ref[i] * 8, 8)
        mol_scr[pl.ds(b8, SLP), :] += c_scr[:SLP, :]

    @pl.when(fl_ref[i] == 0)
    def _slow():
        def sbody(j, _):
            d = dest_ref[base + j]
            mol_scr[pl.ds(d, 1), :] += c_scr[pl.ds(j, 1), :]
            return 0

        jax.lax.fori_loop(0, nd, sbody, 0)

    @pl.when(i == NB - 1)
    def _heads():
        for t in range(T):
            mt = mol_scr[:B, t * D:(t + 1) * D]
            h = jnp.maximum(mt @ fc1w_ref[t] + fc1b_ref[t:t + 1, :], 0.0)
            h = jnp.maximum(h @ fc2w_ref[t] + fc2b_ref[t:t + 1, :], 0.0)
            h = jnp.maximum(h @ fc3w_ref[t] + fc3b_ref[t:t + 1, :], 0.0)
            p = h @ outw_ref[..., t:t + 1] + outb_ref[0, t]
            out_ref[:, t:t + 1] = p


@functools.partial(jax.jit, static_argnames=())
def kernel(node_feats, etype, graph_ids, atom_W, atom_b, fc1_W, fc1_b,
           fc2_W, fc2_b, fc3_W, fc3_b, out_W, out_b):
    del etype
    g = graph_ids.astype(jnp.int32)

    first = jnp.concatenate([jnp.ones((1,), jnp.bool_), g[1:] != g[:-1]])
    f2 = jnp.concatenate(
        [first, jnp.zeros((NB * K - N,), jnp.bool_)]).reshape(NB, K)
    f2 = f2 | (jax.lax.broadcasted_iota(jnp.int32, (NB, K), 1) == 0)
    rc = jnp.cumsum(f2.astype(jnp.int32), axis=1)
    r2 = rc - 1
    nd_b = rc[:, -1]
    off = jnp.concatenate([jnp.zeros((1,), jnp.int32),
                           jnp.cumsum(nd_b)]).astype(jnp.int32)
    slotf = (r2 + off[:-1, None]).reshape(-1)
    dest = g[jnp.clip(jnp.searchsorted(slotf, jnp.arange(L, dtype=jnp.int32),
                                       side="left"), 0, N - 1)]
    bad = (dest[1:] - dest[:-1]) != 1
    cb = jnp.concatenate([jnp.zeros((1,), jnp.int32),
                          jnp.cumsum(bad.astype(jnp.int32))])
    anybad = cb[jnp.maximum(off[1:] - 1, off[:-1])] - cb[off[:-1]]
    fl = ((anybad == 0) & (nd_b <= SL)).astype(jnp.int32)
    d0_b = dest[off[:-1]]
    d0a_b = (d0_b // 8) * 8
    q_b = jnp.where(fl == 1, d0_b - d0a_b, 0)
    off8 = (d0a_b // 8).astype(jnp.int32)
    r3 = (r2 + q_b[:, None]).reshape(NB, 1, K)

    a_pad = jnp.zeros((D, TP), jnp.float32).at[:, :T].set(atom_W[:, :, 0].T)
    ab_pad = jnp.zeros((1, TP), jnp.float32).at[0, :T].set(atom_b[:, 0])
    outw = out_W[:, :, 0].T
    outb = out_b[:, 0][None, :]

    grid_spec = pltpu.PrefetchScalarGridSpec(
        num_scalar_prefetch=4,
        grid=(NB,),
        in_specs=[
            pl.BlockSpec((K, D), lambda i, *_: (i, 0)),
            pl.BlockSpec((1, 1, K), lambda i, *_: (i, 0, 0)),
            pl.BlockSpec((D, TP), lambda i, *_: (0, 0)),
            pl.BlockSpec((1, TP), lambda i, *_: (0, 0)),
            pl.BlockSpec((T, D, H), lambda i, *_: (0, 0, 0)),
            pl.BlockSpec((T, H), lambda i, *_: (0, 0)),
            pl.BlockSpec((T, H, H), lambda i, *_: (0, 0, 0)),
            pl.BlockSpec((T, H), lambda i, *_: (0, 0)),
            pl.BlockSpec((T, H, H), lambda i, *_: (0, 0, 0)),
            pl.BlockSpec((T, H), lambda i, *_: (0, 0)),
            pl.BlockSpec((H, T), lambda i, *_: (0, 0)),
            pl.BlockSpec((1, T), lambda i, *_: (0, 0)),
        ],
        out_specs=pl.BlockSpec((B, T), lambda i, *_: (0, 0)),
        scratch_shapes=[
            pltpu.VMEM((B + SLP, T * D), jnp.float32),
            pltpu.VMEM((K, T * D), jnp.float32),
        ],
    )
    return pl.pallas_call(
        _body,
        grid_spec=grid_spec,
        out_shape=jax.ShapeDtypeStruct((B, T), jnp.float32),
        compiler_params=pltpu.CompilerParams(
            dimension_semantics=("arbitrary",)),
    )(off, dest, fl, off8, node_feats, r3, a_pad, ab_pad,
      fc1_W, fc1_b, fc2_W, fc2_b, fc3_W, fc3_b, outw, outb)
